# Initial kernel scaffold; baseline (speedup 1.0000x reference)
import jax
import jax.numpy as jnp
from jax.experimental import pallas as pl


def kernel(x, edge_index, w, W0, W1, L0w, L0b, L1w, L1b, g0, b0, g1, b1):
    raise NotImplementedError("write your pallas kernel here")


# speedup vs baseline: 5.9037x; 5.9037x over previous
import functools

import jax
import jax.numpy as jnp
from jax import lax
from jax.experimental import pallas as pl
from jax.experimental.pallas import tpu as pltpu
from jax.experimental.pallas import tpu_sc as plsc

N = 10000
D = 128
E = 320000

NC = 2
NS = 16
NW = NC * NS
EPW = E // NW
K = 80
NCHUNK = EPW // K
NPAD = 10240
RPS = NPAD // NS

_mesh = plsc.VectorSubcoreMesh(core_axis_name="c", subcore_axis_name="s")


def _sc_edge_body(y_hbm, src_hbm, dst_hbm, w_hbm, out_hbm,
                  h_sh, src_v, dst_v, w_v, src_c, dst_c, rows_v, sem):
    c = lax.axis_index("c")
    s = lax.axis_index("s")
    base = (c * NS + s) * EPW

    zvec = jnp.zeros((16,), jnp.float32)

    def _zrow(r, carry):
        for j in range(D // 16):
            rows_v[r, pl.ds(16 * j, 16)] = zvec
        return carry

    lax.fori_loop(0, K, _zrow, 0)
    for t in range(RPS // K):
        pltpu.sync_copy(rows_v, h_sh.at[pl.ds(s * RPS + t * K, K)])

    pltpu.sync_copy(src_hbm.at[pl.ds(base, EPW)], src_v)
    pltpu.sync_copy(dst_hbm.at[pl.ds(base, EPW)], dst_v)
    pltpu.sync_copy(w_hbm.at[pl.ds(base, EPW)], w_v)
    plsc.subcore_barrier()

    def _chunk(i, carry):
        for g in range(K // 16):
            src_c[pl.ds(g * 16, 16)] = src_v[pl.ds(i * K + g * 16, 16)]
            dst_c[pl.ds(g * 16, 16)] = dst_v[pl.ds(i * K + g * 16, 16)]
        pltpu.async_copy(y_hbm.at[src_c], rows_v, sem).wait()

        def _grp(g, gcarry):
            wvec = w_v[pl.ds(i * K + g * 16, 16)]
            for lane in range(16):
                wb = jnp.broadcast_to(wvec[lane], (16,))
                e = g * 16 + lane
                for j in range(D // 16):
                    rows_v[e, pl.ds(16 * j, 16)] = (
                        rows_v[e, pl.ds(16 * j, 16)] * wb)
            return gcarry

        lax.fori_loop(0, K // 16, _grp, 0)
        pltpu.sync_copy(rows_v, h_sh.at[dst_c], add=True)
        return carry

    lax.fori_loop(0, NCHUNK, _chunk, 0)
    plsc.subcore_barrier()
    pltpu.sync_copy(h_sh.at[pl.ds(s * RPS, RPS)],
                    out_hbm.at[c, pl.ds(s * RPS, RPS)])


_sc_edge = pl.kernel(
    _sc_edge_body,
    out_type=jax.ShapeDtypeStruct((NC, NPAD, D), jnp.float32),
    mesh=_mesh,
    scratch_types=[
        pltpu.VMEM_SHARED((NPAD, D), jnp.float32),
        pltpu.VMEM((EPW,), jnp.int32),
        pltpu.VMEM((EPW,), jnp.int32),
        pltpu.VMEM((EPW,), jnp.float32),
        pltpu.VMEM((K,), jnp.int32),
        pltpu.VMEM((K,), jnp.int32),
        pltpu.VMEM((K, D), jnp.float32),
        pltpu.SemaphoreType.DMA,
    ],
    name="sc_edge_pass",
)


def _tc_pre_body(x_ref, W_ref, Lw_ref, Lb_ref, y_ref, z_ref):
    xv = x_ref[...]
    y_ref[...] = jnp.dot(xv, W_ref[...], preferred_element_type=jnp.float32)
    z_ref[...] = (jnp.dot(xv, Lw_ref[...], preferred_element_type=jnp.float32)
                  + Lb_ref[...])


_tc_pre = pl.pallas_call(
    _tc_pre_body,
    out_shape=(jax.ShapeDtypeStruct((N, D), jnp.float32),
               jax.ShapeDtypeStruct((N, D), jnp.float32)),
)


def _layer_norm(t, g, b):
    mu = jnp.mean(t, axis=-1, keepdims=True)
    xc = t - mu
    var = jnp.mean(xc * xc, axis=-1, keepdims=True)
    return xc / jnp.sqrt(var + 1e-5) * g + b


def _tc_mid_body(z_ref, hp_ref, g_ref, b_ref, W_ref, Lw_ref, Lb_ref,
                 y_ref, z1_ref):
    t = z_ref[...] + hp_ref[0, :N] + hp_ref[1, :N]
    x1 = _layer_norm(t, g_ref[...], b_ref[...])
    y_ref[...] = jnp.dot(x1, W_ref[...], preferred_element_type=jnp.float32)
    z1_ref[...] = (jnp.dot(x1, Lw_ref[...], preferred_element_type=jnp.float32)
                   + Lb_ref[...])


_tc_mid = pl.pallas_call(
    _tc_mid_body,
    out_shape=(jax.ShapeDtypeStruct((N, D), jnp.float32),
               jax.ShapeDtypeStruct((N, D), jnp.float32)),
)


def _tc_post_body(z_ref, hp_ref, g_ref, b_ref, o_ref):
    t = z_ref[...] + hp_ref[0, :N] + hp_ref[1, :N]
    o_ref[...] = _layer_norm(t, g_ref[...], b_ref[...])


_tc_post = pl.pallas_call(
    _tc_post_body,
    out_shape=jax.ShapeDtypeStruct((N, D), jnp.float32),
)


def kernel(x, edge_index, w, W0, W1, L0w, L0b, L1w, L1b, g0, b0, g1, b1):
    src_f = edge_index[0]
    dst_f = edge_index[1]

    L0b2 = L0b.reshape(1, D)
    L1b2 = L1b.reshape(1, D)
    g02, b02 = g0.reshape(1, D), b0.reshape(1, D)
    g12, b12 = g1.reshape(1, D), b1.reshape(1, D)

    y0, z0 = _tc_pre(x, W0, L0w, L0b2)
    hp0 = _sc_edge(y0, src_f, dst_f, w)
    y1, z1 = _tc_mid(z0, hp0, g02, b02, W1, L1w, L1b2)
    hp1 = _sc_edge(y1, src_f, dst_f, w)
    return _tc_post(z1, hp1, g12, b12)
